# Initial kernel scaffold; baseline (speedup 1.0000x reference)
import jax
import jax.numpy as jnp
from jax.experimental import pallas as pl


def kernel(x, edge_index, W):
    raise NotImplementedError("write your pallas kernel here")


# speedup vs baseline: 14.4821x; 14.4821x over previous
import functools

import jax
import jax.numpy as jnp
from jax import lax
from jax.experimental import pallas as pl
from jax.experimental.pallas import tpu as pltpu
from jax.experimental.pallas import tpu_sc as plsc

N_NODES = 10000
N_EDGES = 320000
F_IN = 128
F_OUT = 64

NC = 2
NS = 16
LANES = 16
NW = NC * NS

CH = 128
E_PAD = 327680
ROWS_PER_TILE = E_PAD // (NW * CH)
N_PAD = 10240
NODES_PER_SUB = N_PAD // NS

_mesh = plsc.VectorSubcoreMesh(core_axis_name="c", subcore_axis_name="s")


@functools.partial(
    pl.kernel,
    out_type=jax.ShapeDtypeStruct((NC, N_PAD), jnp.float32),
    mesh=_mesh,
    compiler_params=pltpu.CompilerParams(use_tc_tiling_on_sc=False),
    scratch_types=[
        pltpu.VMEM((ROWS_PER_TILE, CH), jnp.int32),
        pltpu.VMEM((CH,), jnp.float32),
        pltpu.VMEM_SHARED((N_PAD,), jnp.float32),
    ],
)
def _out_degree_kernel(src_hbm, z1_hbm, deg_hbm, idx_v, ones_v, deg_sp):
    c = lax.axis_index("c")
    s = lax.axis_index("s")
    wid = s * NC + c
    sl = pl.ds(s * NODES_PER_SUB, NODES_PER_SUB)
    pltpu.sync_copy(z1_hbm, deg_sp.at[sl])
    for i in range(CH // LANES):
        ones_v[pl.ds(i * LANES, LANES)] = jnp.ones((LANES,), jnp.float32)
    pltpu.sync_copy(src_hbm.at[pl.ds(wid * ROWS_PER_TILE, ROWS_PER_TILE)], idx_v)
    plsc.subcore_barrier()

    def body(g, carry):
        pltpu.sync_copy(ones_v, deg_sp.at[idx_v.at[g]], add=True)
        return carry

    lax.fori_loop(0, ROWS_PER_TILE, body, 0)
    plsc.subcore_barrier()
    pltpu.sync_copy(deg_sp.at[sl], deg_hbm.at[c].at[sl])


@functools.partial(
    pl.kernel,
    out_type=(
        jax.ShapeDtypeStruct((NC, N_PAD, F_OUT), jnp.float32),
        jax.ShapeDtypeStruct((NC, N_PAD), jnp.float32),
    ),
    mesh=_mesh,
    compiler_params=pltpu.CompilerParams(use_tc_tiling_on_sc=False),
    scratch_types=[
        pltpu.VMEM((ROWS_PER_TILE, CH), jnp.int32),
        pltpu.VMEM((ROWS_PER_TILE, CH), jnp.int32),
        pltpu.VMEM((CH, F_OUT), jnp.float32),
        pltpu.VMEM((CH, F_OUT), jnp.float32),
        pltpu.VMEM((CH,), jnp.float32),
        pltpu.VMEM_SHARED((N_PAD, F_OUT), jnp.float32),
        pltpu.VMEM_SHARED((N_PAD,), jnp.float32),
        pltpu.SemaphoreType.DMA,
        pltpu.SemaphoreType.DMA,
    ],
)
def _aggregate_kernel(y_hbm, src_hbm, dst_hbm, zrow_hbm, z1_hbm,
                      part_hbm, indeg_hbm,
                      idxs_v, idxd_v, rows0, rows1, ones_v,
                      acc_sp, deg_sp, sem0, sem1):
    c = lax.axis_index("c")
    s = lax.axis_index("s")
    wid = s * NC + c
    base = wid * ROWS_PER_TILE
    sl = pl.ds(s * NODES_PER_SUB, NODES_PER_SUB)
    pltpu.sync_copy(zrow_hbm, acc_sp.at[sl])
    pltpu.sync_copy(z1_hbm, deg_sp.at[sl])
    for i in range(CH // LANES):
        ones_v[pl.ds(i * LANES, LANES)] = jnp.ones((LANES,), jnp.float32)
    pltpu.sync_copy(src_hbm.at[pl.ds(base, ROWS_PER_TILE)], idxs_v)
    pltpu.sync_copy(dst_hbm.at[pl.ds(base, ROWS_PER_TILE)], idxd_v)
    plsc.subcore_barrier()

    pltpu.async_copy(y_hbm.at[idxs_v.at[0]], rows0, sem0)
    pltpu.async_copy(y_hbm.at[idxs_v.at[1]], rows1, sem1)

    def body(i, carry):
        g0 = i * 2
        g1 = g0 + 1
        pltpu.make_async_copy(y_hbm.at[idxs_v.at[g0]], rows0, sem0).wait()
        pltpu.sync_copy(rows0, acc_sp.at[idxd_v.at[g0]], add=True)
        pltpu.sync_copy(ones_v, deg_sp.at[idxd_v.at[g0]], add=True)

        @pl.when(g0 + 2 < ROWS_PER_TILE)
        def _():
            pltpu.async_copy(y_hbm.at[idxs_v.at[g0 + 2]], rows0, sem0)

        pltpu.make_async_copy(y_hbm.at[idxs_v.at[g1]], rows1, sem1).wait()
        pltpu.sync_copy(rows1, acc_sp.at[idxd_v.at[g1]], add=True)
        pltpu.sync_copy(ones_v, deg_sp.at[idxd_v.at[g1]], add=True)

        @pl.when(g1 + 2 < ROWS_PER_TILE)
        def _():
            pltpu.async_copy(y_hbm.at[idxs_v.at[g1 + 2]], rows1, sem1)

        return carry

    lax.fori_loop(0, ROWS_PER_TILE // 2, body, 0)
    plsc.subcore_barrier()
    pltpu.sync_copy(acc_sp.at[sl], part_hbm.at[c].at[sl])
    pltpu.sync_copy(deg_sp.at[sl], indeg_hbm.at[c].at[sl])


BLK = 1280


def _matmul_body(deg_ref, x_ref, w_ref, y_ref):
    deg = deg_ref[0, :] + deg_ref[1, :]
    norm = lax.rsqrt(jnp.maximum(deg, 1.0))
    y_ref[...] = jnp.dot(x_ref[...] * norm[:, None], w_ref[...],
                         preferred_element_type=jnp.float32)


_matmul = pl.pallas_call(
    _matmul_body,
    grid=(N_PAD // BLK,),
    in_specs=[
        pl.BlockSpec((2, BLK), lambda i: (0, i)),
        pl.BlockSpec((BLK, F_IN), lambda i: (i, 0)),
        pl.BlockSpec((F_IN, F_OUT), lambda i: (0, 0)),
    ],
    out_specs=pl.BlockSpec((BLK, F_OUT), lambda i: (i, 0)),
    out_shape=jax.ShapeDtypeStruct((N_PAD, F_OUT), jnp.float32),
)


def _finalize_body(part_ref, indeg_ref, out_ref):
    acc = part_ref[0] + part_ref[1]
    deg = indeg_ref[0, :] + indeg_ref[1, :]
    norm = lax.rsqrt(jnp.maximum(deg, 1.0))
    out_ref[...] = acc * norm[:, None]


_finalize = pl.pallas_call(
    _finalize_body,
    grid=(N_PAD // BLK,),
    in_specs=[
        pl.BlockSpec((2, BLK, F_OUT), lambda i: (0, i, 0)),
        pl.BlockSpec((2, BLK), lambda i: (0, i)),
    ],
    out_specs=pl.BlockSpec((BLK, F_OUT), lambda i: (i, 0)),
    out_shape=jax.ShapeDtypeStruct((N_PAD, F_OUT), jnp.float32),
)


@jax.jit
def kernel(x, edge_index, W):
    src = edge_index[0].astype(jnp.int32)
    dst = edge_index[1].astype(jnp.int32)
    n_extra = E_PAD - N_EDGES
    pad = N_NODES + (jnp.arange(n_extra, dtype=jnp.int32) % (N_PAD - N_NODES))
    src_p = jnp.concatenate([src, pad]).reshape(E_PAD // CH, CH)
    dst_p = jnp.concatenate([dst, pad]).reshape(E_PAD // CH, CH)
    zrow = jnp.zeros((NODES_PER_SUB, F_OUT), jnp.float32)
    z1 = jnp.zeros((NODES_PER_SUB,), jnp.float32)

    out_deg = _out_degree_kernel(src_p, z1)
    x_pad = jnp.pad(x, ((0, N_PAD - N_NODES), (0, 0)))
    y = _matmul(out_deg, x_pad, W)
    part, in_deg = _aggregate_kernel(y, src_p, dst_p, zrow, z1)
    out = _finalize(part, in_deg)
    return out[:N_NODES]


# speedup vs baseline: 15.4439x; 1.0664x over previous
import functools

import jax
import jax.numpy as jnp
from jax import lax
from jax.experimental import pallas as pl
from jax.experimental.pallas import tpu as pltpu
from jax.experimental.pallas import tpu_sc as plsc

N_NODES = 10000
N_EDGES = 320000
F_IN = 128
F_OUT = 64

NC = 2
NS = 16
LANES = 16
NW = NC * NS

CH = 128
NROWS = N_EDGES // CH
ROWS_BASE = NROWS // NW
ROWS_EXTRA = NROWS % NW
N_PAD = 10240
DEG_PER_SUB = N_PAD // NS
ACC_PER_SUB = N_NODES // NS

_mesh = plsc.VectorSubcoreMesh(core_axis_name="c", subcore_axis_name="s")
_sc_params = pltpu.CompilerParams(use_tc_tiling_on_sc=False)


def _tile_rows(wid):
    base = ROWS_BASE * wid + jnp.minimum(wid, ROWS_EXTRA)
    n = ROWS_BASE + jnp.where(wid < ROWS_EXTRA, 1, 0)
    return base, n


def _load_tile_rows(src3d, base, wid, idx_v):
    pltpu.sync_copy(src3d.at[pl.ds(base, ROWS_BASE)],
                    idx_v.at[pl.ds(0, ROWS_BASE)])

    @pl.when(wid < ROWS_EXTRA)
    def _():
        pltpu.sync_copy(src3d.at[pl.ds(base + ROWS_BASE, 1)],
                        idx_v.at[pl.ds(ROWS_BASE, 1)])


@functools.partial(
    pl.kernel,
    out_type=jax.ShapeDtypeStruct((NC, N_PAD), jnp.float32),
    mesh=_mesh,
    compiler_params=_sc_params,
    scratch_types=[
        pltpu.VMEM((ROWS_BASE + 1, CH), jnp.int32),
        pltpu.VMEM((CH,), jnp.float32),
        pltpu.VMEM_SHARED((N_PAD,), jnp.float32),
    ],
)
def _out_degree_kernel(e3d_hbm, z1_hbm, deg_hbm, idx_v, ones_v, deg_sp):
    c = lax.axis_index("c")
    s = lax.axis_index("s")
    wid = s * NC + c
    base, nrows = _tile_rows(wid)
    sl = pl.ds(s * DEG_PER_SUB, DEG_PER_SUB)
    pltpu.sync_copy(z1_hbm, deg_sp.at[sl])
    for i in range(CH // LANES):
        ones_v[pl.ds(i * LANES, LANES)] = jnp.ones((LANES,), jnp.float32)
    _load_tile_rows(e3d_hbm.at[0], base, wid, idx_v)
    plsc.subcore_barrier()

    def body(g, carry):
        pltpu.sync_copy(ones_v, deg_sp.at[idx_v.at[g]], add=True)
        return carry

    lax.fori_loop(0, nrows, body, 0)
    plsc.subcore_barrier()
    pltpu.sync_copy(deg_sp.at[sl], deg_hbm.at[c].at[sl])


@functools.partial(
    pl.kernel,
    out_type=(
        jax.ShapeDtypeStruct((NC, N_NODES, F_OUT), jnp.float32),
        jax.ShapeDtypeStruct((NC, N_PAD), jnp.float32),
    ),
    mesh=_mesh,
    compiler_params=_sc_params,
    scratch_types=[
        pltpu.VMEM((ROWS_BASE + 1, CH), jnp.int32),
        pltpu.VMEM((ROWS_BASE + 1, CH), jnp.int32),
        pltpu.VMEM((CH, F_OUT), jnp.float32),
        pltpu.VMEM((CH, F_OUT), jnp.float32),
        pltpu.VMEM((CH,), jnp.float32),
        pltpu.VMEM_SHARED((N_NODES, F_OUT), jnp.float32),
        pltpu.VMEM_SHARED((N_PAD,), jnp.float32),
        pltpu.SemaphoreType.DMA,
        pltpu.SemaphoreType.DMA,
    ],
)
def _aggregate_kernel(y_hbm, e3d_hbm, zrow_hbm, z1_hbm,
                      part_hbm, indeg_hbm,
                      idxs_v, idxd_v, rows0, rows1, ones_v,
                      acc_sp, deg_sp, sem0, sem1):
    c = lax.axis_index("c")
    s = lax.axis_index("s")
    wid = s * NC + c
    base, nrows = _tile_rows(wid)
    sl_acc = pl.ds(s * ACC_PER_SUB, ACC_PER_SUB)
    sl_deg = pl.ds(s * DEG_PER_SUB, DEG_PER_SUB)
    pltpu.sync_copy(zrow_hbm, acc_sp.at[sl_acc])
    pltpu.sync_copy(z1_hbm, deg_sp.at[sl_deg])
    for i in range(CH // LANES):
        ones_v[pl.ds(i * LANES, LANES)] = jnp.ones((LANES,), jnp.float32)
    _load_tile_rows(e3d_hbm.at[0], base, wid, idxs_v)
    _load_tile_rows(e3d_hbm.at[1], base, wid, idxd_v)
    plsc.subcore_barrier()

    pltpu.async_copy(y_hbm.at[idxs_v.at[0]], rows0, sem0)
    pltpu.async_copy(y_hbm.at[idxs_v.at[1]], rows1, sem1)

    def step(g, rows, sem):
        pltpu.make_async_copy(y_hbm.at[idxs_v.at[g]], rows, sem).wait()
        pltpu.sync_copy(rows, acc_sp.at[idxd_v.at[g]], add=True)
        pltpu.sync_copy(ones_v, deg_sp.at[idxd_v.at[g]], add=True)

        @pl.when(g + 2 < nrows)
        def _():
            pltpu.async_copy(y_hbm.at[idxs_v.at[g + 2]], rows, sem)

    def body(i, carry):
        step(i * 2, rows0, sem0)
        step(i * 2 + 1, rows1, sem1)
        return carry

    lax.fori_loop(0, ROWS_BASE // 2, body, 0)

    @pl.when(wid < ROWS_EXTRA)
    def _():
        step(ROWS_BASE, rows0, sem0)

    plsc.subcore_barrier()
    pltpu.sync_copy(acc_sp.at[sl_acc], part_hbm.at[c].at[sl_acc])
    pltpu.sync_copy(deg_sp.at[sl_deg], indeg_hbm.at[c].at[sl_deg])


BLK = 1024


def _matmul_body(deg_ref, x_ref, w_ref, y_ref):
    deg = deg_ref[0, :] + deg_ref[1, :]
    norm = lax.rsqrt(jnp.maximum(deg, 1.0))
    y_ref[...] = jnp.dot(x_ref[...] * norm[:, None], w_ref[...],
                         preferred_element_type=jnp.float32)


_matmul = pl.pallas_call(
    _matmul_body,
    grid=(pl.cdiv(N_NODES, BLK),),
    in_specs=[
        pl.BlockSpec((2, BLK), lambda i: (0, i)),
        pl.BlockSpec((BLK, F_IN), lambda i: (i, 0)),
        pl.BlockSpec((F_IN, F_OUT), lambda i: (0, 0)),
    ],
    out_specs=pl.BlockSpec((BLK, F_OUT), lambda i: (i, 0)),
    out_shape=jax.ShapeDtypeStruct((N_NODES, F_OUT), jnp.float32),
)


def _finalize_body(part_ref, indeg_ref, out_ref):
    acc = part_ref[0] + part_ref[1]
    deg = indeg_ref[0, :] + indeg_ref[1, :]
    norm = lax.rsqrt(jnp.maximum(deg, 1.0))
    out_ref[...] = acc * norm[:, None]


_finalize = pl.pallas_call(
    _finalize_body,
    grid=(pl.cdiv(N_NODES, BLK),),
    in_specs=[
        pl.BlockSpec((2, BLK, F_OUT), lambda i: (0, i, 0)),
        pl.BlockSpec((2, BLK), lambda i: (0, i)),
    ],
    out_specs=pl.BlockSpec((BLK, F_OUT), lambda i: (i, 0)),
    out_shape=jax.ShapeDtypeStruct((N_NODES, F_OUT), jnp.float32),
)


@jax.jit
def kernel(x, edge_index, W):
    e3d = jnp.reshape(edge_index.astype(jnp.int32), (2, NROWS, CH))
    z1 = jnp.zeros((DEG_PER_SUB,), jnp.float32)
    zrow = jnp.zeros((ACC_PER_SUB, F_OUT), jnp.float32)

    out_deg = _out_degree_kernel(e3d, z1)
    y = _matmul(out_deg, x, W)
    part, in_deg = _aggregate_kernel(y, e3d, zrow, z1)
    return _finalize(part, in_deg)


# speedup vs baseline: 16.5460x; 1.0714x over previous
import functools

import jax
import jax.numpy as jnp
from jax import lax
from jax.experimental import pallas as pl
from jax.experimental.pallas import tpu as pltpu
from jax.experimental.pallas import tpu_sc as plsc

N_NODES = 10000
N_EDGES = 320000
F_IN = 128
F_OUT = 64

NC = 2
NS = 16
LANES = 16
NW = NC * NS

CH = 128
NROWS = N_EDGES // CH
ROWS_BASE = NROWS // NW
ROWS_EXTRA = NROWS % NW
N_PAD = 10240
DEG_PER_SUB = N_PAD // NS
ACC_PER_SUB = N_NODES // NS

_mesh = plsc.VectorSubcoreMesh(core_axis_name="c", subcore_axis_name="s")
_sc_params = pltpu.CompilerParams(use_tc_tiling_on_sc=False)


def _tile_rows(wid):
    base = ROWS_BASE * wid + jnp.minimum(wid, ROWS_EXTRA)
    n = ROWS_BASE + jnp.where(wid < ROWS_EXTRA, 1, 0)
    return base, n


def _load_tile_rows(src3d, base, wid, idx_v):
    pltpu.sync_copy(src3d.at[pl.ds(base, ROWS_BASE)],
                    idx_v.at[pl.ds(0, ROWS_BASE)])

    @pl.when(wid < ROWS_EXTRA)
    def _():
        pltpu.sync_copy(src3d.at[pl.ds(base + ROWS_BASE, 1)],
                        idx_v.at[pl.ds(ROWS_BASE, 1)])


@functools.partial(
    pl.kernel,
    out_type=jax.ShapeDtypeStruct((NC, N_PAD), jnp.float32),
    mesh=_mesh,
    compiler_params=_sc_params,
    scratch_types=[
        pltpu.VMEM((ROWS_BASE + 1, CH), jnp.int32),
        pltpu.VMEM((CH,), jnp.float32),
        pltpu.VMEM_SHARED((N_PAD,), jnp.float32),
        pltpu.SemaphoreType.DMA,
    ],
)
def _out_degree_kernel(e3d_hbm, z1_hbm, deg_hbm, idx_v, ones_v, deg_sp, sem):
    c = lax.axis_index("c")
    s = lax.axis_index("s")
    wid = s * NC + c
    base, nrows = _tile_rows(wid)
    sl = pl.ds(s * DEG_PER_SUB, DEG_PER_SUB)
    pltpu.sync_copy(z1_hbm, deg_sp.at[sl])
    for i in range(CH // LANES):
        ones_v[pl.ds(i * LANES, LANES)] = jnp.ones((LANES,), jnp.float32)
    _load_tile_rows(e3d_hbm.at[0], base, wid, idx_v)
    plsc.subcore_barrier()

    def body(g, carry):
        pltpu.async_copy(ones_v, deg_sp.at[idx_v.at[g]], sem, add=True)
        return carry

    lax.fori_loop(0, nrows, body, 0)

    def drain(g, carry):
        pltpu.make_async_copy(ones_v, deg_sp.at[idx_v.at[0]], sem).wait()
        return carry

    lax.fori_loop(0, nrows, drain, 0)
    plsc.subcore_barrier()
    pltpu.sync_copy(deg_sp.at[sl], deg_hbm.at[c].at[sl])


@functools.partial(
    pl.kernel,
    out_type=(
        jax.ShapeDtypeStruct((NC, N_NODES, F_OUT), jnp.float32),
        jax.ShapeDtypeStruct((NC, N_PAD), jnp.float32),
    ),
    mesh=_mesh,
    compiler_params=_sc_params,
    scratch_types=[
        pltpu.VMEM((ROWS_BASE + 1, CH), jnp.int32),
        pltpu.VMEM((ROWS_BASE + 1, CH), jnp.int32),
        [pltpu.VMEM((CH, F_OUT), jnp.float32)] * 4,
        pltpu.VMEM((CH,), jnp.float32),
        pltpu.VMEM_SHARED((N_NODES, F_OUT), jnp.float32),
        pltpu.VMEM_SHARED((N_PAD,), jnp.float32),
        [pltpu.SemaphoreType.DMA] * 4,
        [pltpu.SemaphoreType.DMA] * 4,
        pltpu.SemaphoreType.DMA,
    ],
)
def _aggregate_kernel(y_hbm, e3d_hbm, zrow_hbm, z1_hbm,
                      part_hbm, indeg_hbm,
                      idxs_v, idxd_v, rows, ones_v,
                      acc_sp, deg_sp, sg, ss, semd):
    c = lax.axis_index("c")
    s = lax.axis_index("s")
    wid = s * NC + c
    base, nrows = _tile_rows(wid)
    sl_acc = pl.ds(s * ACC_PER_SUB, ACC_PER_SUB)
    sl_deg = pl.ds(s * DEG_PER_SUB, DEG_PER_SUB)
    pltpu.sync_copy(zrow_hbm, acc_sp.at[sl_acc])
    pltpu.sync_copy(z1_hbm, deg_sp.at[sl_deg])
    for i in range(CH // LANES):
        ones_v[pl.ds(i * LANES, LANES)] = jnp.ones((LANES,), jnp.float32)
    _load_tile_rows(e3d_hbm.at[0], base, wid, idxs_v)
    _load_tile_rows(e3d_hbm.at[1], base, wid, idxd_v)
    plsc.subcore_barrier()

    NBUF = 4
    for b in range(NBUF):
        pltpu.async_copy(y_hbm.at[idxs_v.at[b]], rows[b], sg[b])

    def block(g, b):
        @pl.when(g < nrows)
        def _():
            pltpu.make_async_copy(y_hbm.at[idxs_v.at[g]], rows[b], sg[b]).wait()
            pltpu.async_copy(rows[b], acc_sp.at[idxd_v.at[g]], ss[b], add=True)
            pltpu.async_copy(ones_v, deg_sp.at[idxd_v.at[g]], semd, add=True)
            j = g - 2
            jb = (b - 2) % NBUF

            @pl.when(jnp.logical_and(j >= 0, j + NBUF < nrows))
            def _():
                pltpu.make_async_copy(rows[jb], acc_sp.at[idxd_v.at[0]],
                                      ss[jb]).wait()
                pltpu.async_copy(y_hbm.at[idxs_v.at[j + NBUF]], rows[jb],
                                 sg[jb])

    def body(i, carry):
        for b in range(NBUF):
            block(i * NBUF + b, b)
        return carry

    lax.fori_loop(0, (ROWS_BASE + 1 + NBUF - 1) // NBUF, body, 0)

    for b in range(NBUF):
        pltpu.make_async_copy(rows[b], acc_sp.at[idxd_v.at[0]], ss[b]).wait()

    def drain(g, carry):
        pltpu.make_async_copy(ones_v, deg_sp.at[idxd_v.at[0]], semd).wait()
        return carry

    lax.fori_loop(0, nrows, drain, 0)
    plsc.subcore_barrier()
    pltpu.sync_copy(acc_sp.at[sl_acc], part_hbm.at[c].at[sl_acc])
    pltpu.sync_copy(deg_sp.at[sl_deg], indeg_hbm.at[c].at[sl_deg])


BLK = 1024


def _matmul_body(deg_ref, x_ref, w_ref, y_ref):
    deg = deg_ref[0, :] + deg_ref[1, :]
    norm = lax.rsqrt(jnp.maximum(deg, 1.0))
    y_ref[...] = jnp.dot(x_ref[...] * norm[:, None], w_ref[...],
                         preferred_element_type=jnp.float32)


_matmul = pl.pallas_call(
    _matmul_body,
    grid=(pl.cdiv(N_NODES, BLK),),
    in_specs=[
        pl.BlockSpec((2, BLK), lambda i: (0, i)),
        pl.BlockSpec((BLK, F_IN), lambda i: (i, 0)),
        pl.BlockSpec((F_IN, F_OUT), lambda i: (0, 0)),
    ],
    out_specs=pl.BlockSpec((BLK, F_OUT), lambda i: (i, 0)),
    out_shape=jax.ShapeDtypeStruct((N_NODES, F_OUT), jnp.float32),
)


def _finalize_body(part_ref, indeg_ref, out_ref):
    acc = part_ref[0] + part_ref[1]
    deg = indeg_ref[0, :] + indeg_ref[1, :]
    norm = lax.rsqrt(jnp.maximum(deg, 1.0))
    out_ref[...] = acc * norm[:, None]


_finalize = pl.pallas_call(
    _finalize_body,
    grid=(pl.cdiv(N_NODES, BLK),),
    in_specs=[
        pl.BlockSpec((2, BLK, F_OUT), lambda i: (0, i, 0)),
        pl.BlockSpec((2, BLK), lambda i: (0, i)),
    ],
    out_specs=pl.BlockSpec((BLK, F_OUT), lambda i: (i, 0)),
    out_shape=jax.ShapeDtypeStruct((N_NODES, F_OUT), jnp.float32),
)


@jax.jit
def kernel(x, edge_index, W):
    e3d = jnp.reshape(edge_index.astype(jnp.int32), (2, NROWS, CH))
    z1 = jnp.zeros((DEG_PER_SUB,), jnp.float32)
    zrow = jnp.zeros((ACC_PER_SUB, F_OUT), jnp.float32)

    out_deg = _out_degree_kernel(e3d, z1)
    y = _matmul(out_deg, x, W)
    part, in_deg = _aggregate_kernel(y, e3d, zrow, z1)
    return _finalize(part, in_deg)
